# baseline (speedup 1.0000x reference)
import jax
import jax.numpy as jnp
from jax.experimental import pallas as pl
from jax.experimental.pallas import tpu as pltpu

B = 1024
D = 128
NW = 64
UB = 64


NSLOT = 3
B2 = B // 2
FC = 64


def _body(si_ref, sy_ref, ix_ref, states_ref, w_ref, b_ref,
          out_ref, acc, xbuf, wbuf, bgbuf, stage, stage2, sem, osem):
    g = pl.program_id(0)
    nblk = pl.num_programs(0)

    def issue(blk, slot):
        base = blk * UB
        for k in range(UB):
            si = si_ref[base + k]
            sy = sy_ref[base + k]
            pltpu.make_async_copy(states_ref.at[si], xbuf.at[slot, k],
                                  sem.at[slot]).start()
            pltpu.make_async_copy(w_ref.at[sy], wbuf.at[slot, k],
                                  sem.at[slot]).start()
            pltpu.make_async_copy(b_ref.at[sy], bgbuf.at[slot, k],
                                  sem.at[slot]).start()

    @pl.when(g == 0)
    def _init():
        acc[...] = jnp.zeros_like(acc)
        issue(0, 0)
        issue(1, 1)

    @pl.when(g + 2 < nblk)
    def _prefetch():
        issue(g + 2, (g + 2) % NSLOT)

    slot = g % NSLOT
    pltpu.make_async_copy(states_ref.at[pl.ds(0, UB)], xbuf.at[slot],
                          sem.at[slot]).wait()
    pltpu.make_async_copy(w_ref.at[pl.ds(0, UB)], wbuf.at[slot],
                          sem.at[slot]).wait()
    pltpu.make_async_copy(b_ref.at[pl.ds(0, UB)], bgbuf.at[slot],
                          sem.at[slot]).wait()

    for k in range(UB):
        w = wbuf[slot, k]
        xk = xbuf[slot, k].astype(jnp.bfloat16)
        yt = jax.lax.dot_general(xk, w, (((0,), (1,)), ((), ())),
                                 preferred_element_type=jnp.float32)
        yt = yt + bgbuf[slot, k]
        sq = jnp.sum(yt * yt, axis=1, keepdims=True)
        yt = yt * jax.lax.rsqrt(jnp.maximum(sq, 1e-12))
        i = ix_ref[g * UB + k]
        acc[pl.ds(i, 1)] = acc[pl.ds(i, 1)] + yt[None]

    @pl.when(g == nblk - 1)
    def _flush():
        for ci in range(B2 // FC):
            eye = (jax.lax.broadcasted_iota(jnp.int32, (NW, NW), 0)
                   == jax.lax.broadcasted_iota(jnp.int32, (NW, NW), 1)
                   ).astype(jnp.bfloat16)

            def pair(j, _):
                c = ci * FC + j
                lo = acc[pl.ds(c, 1)][0].astype(jnp.bfloat16)
                hi = acc[pl.ds(c + B2, 1)][0].astype(jnp.bfloat16)
                stage[pl.ds(j, 1)] = jax.lax.dot_general(
                    lo, eye, (((0,), (0,)), ((), ())),
                    preferred_element_type=jnp.float32)[None]
                stage2[pl.ds(j, 1)] = jax.lax.dot_general(
                    hi, eye, (((0,), (0,)), ((), ())),
                    preferred_element_type=jnp.float32)[None]
                return 0
            jax.lax.fori_loop(0, FC, pair, 0)
            cp0 = pltpu.make_async_copy(
                stage, out_ref.at[0, pl.ds(ci * FC, FC)], osem)
            cp1 = pltpu.make_async_copy(
                stage2, out_ref.at[1, pl.ds(ci * FC, FC)], osem)
            cp0.start()
            cp1.start()
            cp0.wait()
            cp1.wait()


def kernel(computed_states, W, b, indices, symbols, args):
    U = indices.shape[0]
    stacked_index = args[:, 0] * B + indices
    w16 = W.astype(jnp.bfloat16)
    b2 = b[:, None, :]

    out = pl.pallas_call(
        _body,
        grid_spec=pltpu.PrefetchScalarGridSpec(
            num_scalar_prefetch=3,
            grid=(U // UB,),
            in_specs=[
                pl.BlockSpec(memory_space=pltpu.MemorySpace.HBM),
                pl.BlockSpec(memory_space=pltpu.MemorySpace.HBM),
                pl.BlockSpec(memory_space=pltpu.MemorySpace.HBM),
            ],
            out_specs=pl.BlockSpec(memory_space=pltpu.MemorySpace.HBM),
            scratch_shapes=[
                pltpu.VMEM((B, NW, D), jnp.float32),
                pltpu.VMEM((NSLOT, UB, D, NW), jnp.float32),
                pltpu.VMEM((NSLOT, UB, D, D), jnp.bfloat16),
                pltpu.VMEM((NSLOT, UB, 1, D), jnp.float32),
                pltpu.VMEM((FC, D, NW), jnp.float32),
                pltpu.VMEM((FC, D, NW), jnp.float32),
                pltpu.SemaphoreType.DMA((NSLOT,)),
                pltpu.SemaphoreType.DMA,
            ],
        ),
        out_shape=jax.ShapeDtypeStruct((2, B2, D, NW), jnp.float32),
        compiler_params=pltpu.CompilerParams(
            dimension_semantics=("arbitrary",),
            vmem_limit_bytes=100 * 1024 * 1024,
        ),
    )(stacked_index, symbols, indices, computed_states, w16, b2)
    return out.reshape(B, D, NW)


# speedup vs baseline: 1.0317x; 1.0317x over previous
import jax
import jax.numpy as jnp
from jax.experimental import pallas as pl
from jax.experimental.pallas import tpu as pltpu

B = 1024
D = 128
NW = 64
UB = 64


NSLOT = 3
B4 = B // 4
FC = 32


def _body(si_ref, sy_ref, ix_ref, states_ref, w_ref, b_ref,
          out_ref, acc, xbuf, wbuf, bgbuf, stage, sem, osem):
    g = pl.program_id(0)
    nblk = pl.num_programs(0)

    def issue(blk, slot):
        base = blk * UB
        for k in range(UB):
            si = si_ref[base + k]
            sy = sy_ref[base + k]
            pltpu.make_async_copy(states_ref.at[si], xbuf.at[slot, k],
                                  sem.at[slot]).start()
            pltpu.make_async_copy(w_ref.at[sy], wbuf.at[slot, k],
                                  sem.at[slot]).start()
            pltpu.make_async_copy(b_ref.at[sy], bgbuf.at[slot, k],
                                  sem.at[slot]).start()

    @pl.when(g == 0)
    def _init():
        acc[...] = jnp.zeros_like(acc)
        issue(0, 0)
        issue(1, 1)

    @pl.when(g + 2 < nblk)
    def _prefetch():
        issue(g + 2, (g + 2) % NSLOT)

    slot = g % NSLOT
    pltpu.make_async_copy(states_ref.at[pl.ds(0, UB)], xbuf.at[slot],
                          sem.at[slot]).wait()
    pltpu.make_async_copy(w_ref.at[pl.ds(0, UB)], wbuf.at[slot],
                          sem.at[slot]).wait()
    pltpu.make_async_copy(b_ref.at[pl.ds(0, UB)], bgbuf.at[slot],
                          sem.at[slot]).wait()

    for k in range(UB):
        w = wbuf[slot, k]
        xk = xbuf[slot, k].astype(jnp.bfloat16)
        yt = jax.lax.dot_general(xk, w, (((0,), (1,)), ((), ())),
                                 preferred_element_type=jnp.float32)
        yt = yt + bgbuf[slot, k]
        sq = jnp.sum(yt * yt, axis=1, keepdims=True)
        yt = yt * jax.lax.rsqrt(jnp.maximum(sq, 1e-12))
        i = ix_ref[g * UB + k]
        acc[pl.ds(i, 1)] = acc[pl.ds(i, 1)] + yt[None]

    @pl.when(g == nblk - 1)
    def _flush():
        eye = (jax.lax.broadcasted_iota(jnp.int32, (4 * NW, 4 * NW), 0)
               == jax.lax.broadcasted_iota(jnp.int32, (4 * NW, 4 * NW), 1)
               ).astype(jnp.bfloat16)
        for ci in range(B4 // FC):
            def quad(j, _):
                c = ci * FC + j
                v = jnp.concatenate(
                    [acc[pl.ds(c + q * B4, 1)][0] for q in range(4)],
                    axis=0).astype(jnp.bfloat16)
                tt = jax.lax.dot_general(v, eye, (((0,), (0,)), ((), ())),
                                         preferred_element_type=jnp.float32)
                stage[pl.ds(j, 1)] = tt.reshape(D, 4, NW)[None]
                return 0
            jax.lax.fori_loop(0, FC, quad, 0)
            cps = [pltpu.make_async_copy(
                stage.at[:, :, q], out_ref.at[q, pl.ds(ci * FC, FC)], osem)
                for q in range(4)]
            for cp in cps:
                cp.start()
            for cp in cps:
                cp.wait()


def kernel(computed_states, W, b, indices, symbols, args):
    U = indices.shape[0]
    stacked_index = args[:, 0] * B + indices
    w16 = W.astype(jnp.bfloat16)
    b2 = b[:, None, :]

    out = pl.pallas_call(
        _body,
        grid_spec=pltpu.PrefetchScalarGridSpec(
            num_scalar_prefetch=3,
            grid=(U // UB,),
            in_specs=[
                pl.BlockSpec(memory_space=pltpu.MemorySpace.HBM),
                pl.BlockSpec(memory_space=pltpu.MemorySpace.HBM),
                pl.BlockSpec(memory_space=pltpu.MemorySpace.HBM),
            ],
            out_specs=pl.BlockSpec(memory_space=pltpu.MemorySpace.HBM),
            scratch_shapes=[
                pltpu.VMEM((B, NW, D), jnp.float32),
                pltpu.VMEM((NSLOT, UB, D, NW), jnp.float32),
                pltpu.VMEM((NSLOT, UB, D, D), jnp.bfloat16),
                pltpu.VMEM((NSLOT, UB, 1, D), jnp.float32),
                pltpu.VMEM((FC, D, 4, NW), jnp.float32),
                pltpu.SemaphoreType.DMA((NSLOT,)),
                pltpu.SemaphoreType.DMA,
            ],
        ),
        out_shape=jax.ShapeDtypeStruct((4, B4, D, NW), jnp.float32),
        compiler_params=pltpu.CompilerParams(
            dimension_semantics=("arbitrary",),
            vmem_limit_bytes=100 * 1024 * 1024,
        ),
    )(stacked_index, symbols, indices, computed_states, w16, b2)
    return out.reshape(B, D, NW)


# speedup vs baseline: 1.3264x; 1.2857x over previous
import jax
import jax.numpy as jnp
from jax.experimental import pallas as pl
from jax.experimental.pallas import tpu as pltpu

B = 1024
D = 128
NW = 64
UB = 64


NSLOT = 3


def _body(si_ref, sy_ref, ix_ref, states_ref, w_ref, b_ref,
          out_ref, acc, xbuf, wbuf, bgbuf, sem, osem):
    g = pl.program_id(0)
    nblk = pl.num_programs(0)

    def issue(blk, slot):
        base = blk * UB
        for k in range(UB):
            si = si_ref[base + k]
            sy = sy_ref[base + k]
            pltpu.make_async_copy(states_ref.at[si], xbuf.at[slot, k],
                                  sem.at[slot]).start()
            pltpu.make_async_copy(w_ref.at[sy], wbuf.at[slot, k],
                                  sem.at[slot]).start()
            pltpu.make_async_copy(b_ref.at[sy], bgbuf.at[slot, k],
                                  sem.at[slot]).start()

    @pl.when(g == 0)
    def _init():
        acc[...] = jnp.zeros_like(acc)
        issue(0, 0)
        issue(1, 1)

    @pl.when(g + 2 < nblk)
    def _prefetch():
        issue(g + 2, (g + 2) % NSLOT)

    slot = g % NSLOT
    pltpu.make_async_copy(states_ref.at[pl.ds(0, UB)], xbuf.at[slot],
                          sem.at[slot]).wait()
    pltpu.make_async_copy(w_ref.at[pl.ds(0, UB)], wbuf.at[slot],
                          sem.at[slot]).wait()
    pltpu.make_async_copy(b_ref.at[pl.ds(0, UB)], bgbuf.at[slot],
                          sem.at[slot]).wait()

    for k in range(UB):
        w = wbuf[slot, k]
        xk = xbuf[slot, k].astype(jnp.bfloat16)
        yt = jax.lax.dot_general(xk, w, (((0,), (1,)), ((), ())),
                                 preferred_element_type=jnp.float32)
        yt = yt + bgbuf[slot, k]
        sq = jnp.sum(yt * yt, axis=1, keepdims=True)
        yt = yt * jax.lax.rsqrt(jnp.maximum(sq, 1e-12))
        i = ix_ref[g * UB + k]
        acc[pl.ds(i, 1)] = acc[pl.ds(i, 1)] + yt[None]

    @pl.when(g == nblk - 1)
    def _flush():
        pltpu.make_async_copy(acc, out_ref, osem).start()
        pltpu.make_async_copy(acc, out_ref, osem).wait()


def kernel(computed_states, W, b, indices, symbols, args):
    U = indices.shape[0]
    stacked_index = args[:, 0] * B + indices
    w16 = W.astype(jnp.bfloat16)
    b2 = b[:, None, :]

    out = pl.pallas_call(
        _body,
        grid_spec=pltpu.PrefetchScalarGridSpec(
            num_scalar_prefetch=3,
            grid=(U // UB,),
            in_specs=[
                pl.BlockSpec(memory_space=pltpu.MemorySpace.HBM),
                pl.BlockSpec(memory_space=pltpu.MemorySpace.HBM),
                pl.BlockSpec(memory_space=pltpu.MemorySpace.HBM),
            ],
            out_specs=pl.BlockSpec(memory_space=pltpu.MemorySpace.HBM),
            scratch_shapes=[
                pltpu.VMEM((B, NW, D), jnp.float32),
                pltpu.VMEM((NSLOT, UB, D, NW), jnp.float32),
                pltpu.VMEM((NSLOT, UB, D, D), jnp.bfloat16),
                pltpu.VMEM((NSLOT, UB, 1, D), jnp.float32),
                pltpu.SemaphoreType.DMA((NSLOT,)),
                pltpu.SemaphoreType.DMA,
            ],
        ),
        out_shape=jax.ShapeDtypeStruct((B, NW, D), jnp.float32),
        compiler_params=pltpu.CompilerParams(
            dimension_semantics=("arbitrary",),
            vmem_limit_bytes=100 * 1024 * 1024,
        ),
    )(stacked_index, symbols, indices, computed_states, w16, b2)
    return jnp.swapaxes(out, 1, 2)
